# baseline (speedup 1.0000x reference)
import functools

import jax
import jax.numpy as jnp
from jax import lax
from jax.experimental import pallas as pl
from jax.experimental.pallas import tpu as pltpu
from jax.experimental.pallas import tpu_sc as plsc

_NC = 2
_NS = 16
_NW = _NC * _NS
_CH = 128


def _mesh():
    return plsc.VectorSubcoreMesh(core_axis_name="c", subcore_axis_name="s",
                                  num_cores=_NC, num_subcores=_NS)


def _make_seg(n_nodes, n_feat, n_chunks_pad):
    npad = ((n_nodes + _CH * _NS - 1) // (_CH * _NS)) * (_CH * _NS)
    rows_per_sub = npad // _NS
    blocks_per_sub = rows_per_sub // _CH
    tpb = n_chunks_pad // _NW
    nblk = tpb // 8
    assert nblk % 2 == 0

    def body(x_hbm, gidx_hbm, sidx_hbm, sums_hbm, cnt_hbm,
             gi0, gi1, si0, si1, rows0, rows1, onesbuf, acc_sh, cnt_sh,
             semi0, semi1, semg0, semg1, sems0, sems1):
        c = lax.axis_index("c")
        s = lax.axis_index("s")
        w = s * _NC + c
        gi = (gi0, gi1)
        si = (si0, si1)
        rows = (rows0, rows1)
        semi = (semi0, semi1)
        semg = (semg0, semg1)
        sems = (sems0, sems1)

        def zrow(r, carry):
            for k in range(n_feat // 16):
                rows0[r, pl.ds(k * 16, 16)] = jnp.zeros((16,), jnp.float32)
            return carry
        lax.fori_loop(0, _CH, zrow, 0)
        for k in range(_CH // 16):
            onesbuf[pl.ds(k * 16, 16)] = jnp.ones((16,), jnp.float32)

        for j in range(blocks_per_sub):
            r0 = s * rows_per_sub + j * _CH
            pltpu.sync_copy(rows0, acc_sh.at[pl.ds(r0, _CH)])
            pltpu.sync_copy(rows0.at[0], cnt_sh.at[pl.ds(r0, _CH)])
        plsc.subcore_barrier()

        def fire_idx(slot, blk):
            o = pl.multiple_of(blk * 8, 8)
            pltpu.async_copy(gidx_hbm.at[w, pl.ds(o, 8)], gi[slot],
                             semi[slot])
            pltpu.async_copy(sidx_hbm.at[w, pl.ds(o, 8)], si[slot],
                             semi[slot])

        def drain_idx(slot):
            pltpu.make_async_copy(gidx_hbm.at[0, pl.ds(0, 8)], gi[slot],
                                  semi[slot]).wait()
            pltpu.make_async_copy(sidx_hbm.at[0, pl.ds(0, 8)], si[slot],
                                  semi[slot]).wait()

        def fire_gathers(p, idxrow):
            pltpu.async_copy(x_hbm.at[idxrow], rows[p], semg[p])

        def drain_gathers(p):
            pltpu.make_async_copy(x_hbm.at[gi0.at[0]], rows[p],
                                  semg[p]).wait()

        def fire_scatters(p, idxrow):
            pltpu.async_copy(rows[p], acc_sh.at[idxrow], sems[p], add=True)
            pltpu.async_copy(onesbuf, cnt_sh.at[idxrow], sems[p], add=True)

        def drain_scatters(p):
            pltpu.make_async_copy(rows[p], acc_sh.at[si0.at[0]],
                                  sems[p]).wait()
            pltpu.make_async_copy(onesbuf, cnt_sh.at[si0.at[0]],
                                  sems[p]).wait()

        fire_idx(0, 0)
        drain_idx(0)
        fire_gathers(0, gi0.at[0])

        def step(bb, carry):
            for qq in (0, 1):
                blk = 2 * bb + qq
                giq = gi[qq]
                siq = si[qq]
                for r in range(8):
                    p = r % 2
                    drain_gathers(p)
                    fire_scatters(p, siq.at[r])
                    if r == 0:
                        @pl.when(blk >= 1)
                        def _():
                            drain_scatters(1 - p)

                        @pl.when(blk + 1 < nblk)
                        def _():
                            fire_idx(1 - qq, blk + 1)
                    else:
                        drain_scatters(1 - p)
                    if r < 7:
                        fire_gathers(1 - p, giq.at[r + 1])
                    else:
                        @pl.when(blk + 1 < nblk)
                        def _():
                            drain_idx(1 - qq)
                            fire_gathers(1 - p, gi[1 - qq].at[0])
            return carry
        lax.fori_loop(0, nblk // 2, step, 0)
        drain_scatters(1)
        plsc.subcore_barrier()

        for j in range(blocks_per_sub):
            r0 = s * rows_per_sub + j * _CH
            pltpu.sync_copy(acc_sh.at[pl.ds(r0, _CH)], rows0)
            pltpu.sync_copy(rows0, sums_hbm.at[c, pl.ds(r0, _CH)])
            pltpu.sync_copy(cnt_sh.at[pl.ds(r0, _CH)], onesbuf)
            pltpu.sync_copy(onesbuf, cnt_hbm.at[c, pl.ds(r0, _CH)])

    call = pl.kernel(
        body,
        out_type=(
            jax.ShapeDtypeStruct((_NC, npad, n_feat), jnp.float32),
            jax.ShapeDtypeStruct((_NC, npad), jnp.float32),
        ),
        mesh=_mesh(),
        scratch_types=(
            pltpu.VMEM((8, _CH), jnp.int32),
            pltpu.VMEM((8, _CH), jnp.int32),
            pltpu.VMEM((8, _CH), jnp.int32),
            pltpu.VMEM((8, _CH), jnp.int32),
            pltpu.VMEM((_CH, n_feat), jnp.float32),
            pltpu.VMEM((_CH, n_feat), jnp.float32),
            pltpu.VMEM((_CH,), jnp.float32),
            pltpu.VMEM_SHARED((npad, n_feat), jnp.float32),
            pltpu.VMEM_SHARED((npad,), jnp.float32),
            pltpu.SemaphoreType.DMA,
            pltpu.SemaphoreType.DMA,
            pltpu.SemaphoreType.DMA,
            pltpu.SemaphoreType.DMA,
            pltpu.SemaphoreType.DMA,
            pltpu.SemaphoreType.DMA,
        ),
        name="seg_sum_sc",
    )
    return call, npad


def _post_body(n_nodes, proj, sums_ref, cnt_ref, xdst_ref, wl_ref, bl_ref,
               wr_ref, g_ref, be_ref, *rest):
    if proj:
        wh_ref, bh_ref, out_ref = rest
    else:
        (out_ref,) = rest
    s = sums_ref[0] + sums_ref[1]
    s = s[:n_nodes, :]
    cnt = cnt_ref[0] + cnt_ref[1]
    inv = 1.0 / jnp.maximum(cnt[:n_nodes, :], 1.0)
    mean = s * inv
    y = (jnp.dot(mean, wl_ref[...], preferred_element_type=jnp.float32)
         + bl_ref[...]
         + jnp.dot(xdst_ref[...], wr_ref[...],
                   preferred_element_type=jnp.float32))
    r = jnp.maximum(y, 0.0)
    mu = jnp.mean(r, axis=0, keepdims=True)
    var = jnp.mean((r - mu) * (r - mu), axis=0, keepdims=True)
    feat = (r - mu) * lax.rsqrt(var + 1e-5) * g_ref[...] + be_ref[...]
    if proj:
        out_ref[:n_nodes, :] = (jnp.dot(feat, wh_ref[...],
                                        preferred_element_type=jnp.float32)
                                + bh_ref[...])
    else:
        out_ref[...] = feat


def _post(sums, cnt3, xdst, wl, bl, wr, gam, bet):
    n = xdst.shape[0]
    h = wl.shape[1]
    return pl.pallas_call(
        functools.partial(_post_body, n, False),
        out_shape=jax.ShapeDtypeStruct((n, h), jnp.float32),
    )(sums, cnt3, xdst, wl, bl, wr, gam, bet)


def _post_proj(sums, cnt3, xdst, wl, bl, wr, gam, bet, wh, bh,
               nrows_out=None):
    n = xdst.shape[0]
    h = wh.shape[1]
    return pl.pallas_call(
        functools.partial(_post_body, n, True),
        out_shape=jax.ShapeDtypeStruct((nrows_out or n, h), jnp.float32),
    )(sums, cnt3, xdst, wl, bl, wr, gam, bet, wh, bh)


def _make_edge(n_feat, n_chunks_pad, npad):
    _CB = 64
    tpb = n_chunks_pad // _NW
    nch = tpb * 2
    cpb = 16
    nblk = nch // cpb
    assert nblk % 2 == 0
    stage_rows = npad // _NS
    assert stage_rows % _CB == 0

    def body(a_hbm, b_hbm, src_hbm, dst_hbm, w_hbm, bv_hbm, out_hbm,
             si0, si1, di0, di1, abuf0, abuf1, bbuf0, bbuf1, wbuf, b16buf,
             outbuf0, outbuf1, a_sh, semi0, semi1, sema0, sema1,
             semb0, semb1, semo0, semo1):
        c = lax.axis_index("c")
        s = lax.axis_index("s")
        w = s * _NC + c
        si = (si0, si1)
        di = (di0, di1)
        abuf = (abuf0, abuf1)
        bbuf = (bbuf0, bbuf1)
        outbuf = (outbuf0, outbuf1)
        semi = (semi0, semi1)
        sema = (sema0, sema1)
        semb = (semb0, semb1)
        semo = (semo0, semo1)

        for piece in range(stage_rows // _CB):
            r0 = s * stage_rows + piece * _CB
            pltpu.sync_copy(a_hbm.at[pl.ds(r0, _CB)], abuf0)
            pltpu.sync_copy(abuf0, a_sh.at[pl.ds(r0, _CB)])
        pltpu.sync_copy(w_hbm, wbuf)
        pltpu.sync_copy(bv_hbm, b16buf)
        bv = b16buf[...]
        wv = [wbuf[pl.ds(k * 16, 16)] for k in range(n_feat // 16)]
        lanes = lax.iota(jnp.int32, 16)
        plsc.subcore_barrier()

        def fire_idx(slot, blk):
            o = pl.multiple_of(blk * 8, 8)
            pltpu.async_copy(src_hbm.at[w, pl.ds(o, 8)], si[slot],
                             semi[slot])
            pltpu.async_copy(dst_hbm.at[w, pl.ds(o, 8)], di[slot],
                             semi[slot])

        def drain_idx(slot):
            pltpu.make_async_copy(src_hbm.at[0, pl.ds(0, 8)], si[slot],
                                  semi[slot]).wait()
            pltpu.make_async_copy(dst_hbm.at[0, pl.ds(0, 8)], di[slot],
                                  semi[slot]).wait()

        def fire_g(p, sslice, dslice):
            pltpu.async_copy(a_sh.at[sslice], abuf[p], sema[p])
            pltpu.async_copy(b_hbm.at[dslice], bbuf[p], semb[p])

        def drain_g(p):
            pltpu.make_async_copy(a_sh.at[si0.at[0, pl.ds(0, _CB)]],
                                  abuf[p], sema[p]).wait()
            pltpu.make_async_copy(b_hbm.at[di0.at[0, pl.ds(0, _CB)]],
                                  bbuf[p], semb[p]).wait()

        def drain_out(p):
            pltpu.make_async_copy(outbuf[p], out_hbm.at[pl.ds(0, _CB)],
                                  semo[p]).wait()

        def compute(ab, bb, ob):
            def group(gi, carry2):
                r = jnp.zeros((16,), jnp.float32)
                for j in range(16):
                    e = gi * 16 + j
                    acc = bv
                    for k in range(n_feat // 16):
                        av = ab[e, pl.ds(k * 16, 16)]
                        bbv = bb[e, pl.ds(k * 16, 16)]
                        acc = acc + jnp.maximum(av + bbv, 0.0) * wv[k]
                    for sh in (8, 4, 2, 1):
                        acc = acc + acc.at[lanes ^ sh].get(
                            mode="promise_in_bounds", unique_indices=True)
                    r = jnp.where(lanes == j, acc, r)
                ob[pl.ds(gi * 16, 16)] = r
                return carry2
            lax.fori_loop(0, _CB // 16, group, 0)

        fire_idx(0, 0)
        drain_idx(0)
        fire_g(0, si0.at[0, pl.ds(0, _CB)], di0.at[0, pl.ds(0, _CB)])

        def blk_pairs(bb2, carry):
            for qq in (0, 1):
                blk = 2 * bb2 + qq
                siq, diq = si[qq], di[qq]

                def pair(cc2, carry2):
                    for p in (0, 1):
                        t = blk * cpb + 2 * cc2 + p
                        drain_g(p)

                        @pl.when(t >= 2)
                        def _():
                            drain_out(p)

                        if p == 0:
                            @pl.when((cc2 == 0) & (blk + 1 < nblk))
                            def _():
                                fire_idx(1 - qq, blk + 1)
                            fire_g(1, siq.at[cc2, pl.ds(_CB, _CB)],
                                   diq.at[cc2, pl.ds(_CB, _CB)])
                        else:
                            @pl.when(cc2 < cpb // 2 - 1)
                            def _():
                                fire_g(0, siq.at[cc2 + 1, pl.ds(0, _CB)],
                                       diq.at[cc2 + 1, pl.ds(0, _CB)])

                            @pl.when((cc2 == cpb // 2 - 1)
                                     & (blk + 1 < nblk))
                            def _():
                                drain_idx(1 - qq)
                                fire_g(0, si[1 - qq].at[0, pl.ds(0, _CB)],
                                       di[1 - qq].at[0, pl.ds(0, _CB)])

                        compute(abuf[p], bbuf[p], outbuf[p])
                        pltpu.async_copy(
                            outbuf[p],
                            out_hbm.at[pl.ds((w * nch + t) * _CB, _CB)],
                            semo[p])
                    return carry2
                lax.fori_loop(0, cpb // 2, pair, 0)
            return carry
        lax.fori_loop(0, nblk // 2, blk_pairs, 0)
        drain_out(0)
        drain_out(1)

    return pl.kernel(
        body,
        out_type=jax.ShapeDtypeStruct((n_chunks_pad * _CH,), jnp.float32),
        mesh=_mesh(),
        scratch_types=(
            pltpu.VMEM((8, _CH), jnp.int32),
            pltpu.VMEM((8, _CH), jnp.int32),
            pltpu.VMEM((8, _CH), jnp.int32),
            pltpu.VMEM((8, _CH), jnp.int32),
            pltpu.VMEM((_CB, n_feat), jnp.float32),
            pltpu.VMEM((_CB, n_feat), jnp.float32),
            pltpu.VMEM((_CB, n_feat), jnp.float32),
            pltpu.VMEM((_CB, n_feat), jnp.float32),
            pltpu.VMEM((n_feat,), jnp.float32),
            pltpu.VMEM((16,), jnp.float32),
            pltpu.VMEM((_CB,), jnp.float32),
            pltpu.VMEM((_CB,), jnp.float32),
            pltpu.VMEM_SHARED((npad, n_feat), jnp.float32),
            pltpu.SemaphoreType.DMA,
            pltpu.SemaphoreType.DMA,
            pltpu.SemaphoreType.DMA,
            pltpu.SemaphoreType.DMA,
            pltpu.SemaphoreType.DMA,
            pltpu.SemaphoreType.DMA,
            pltpu.SemaphoreType.DMA,
            pltpu.SemaphoreType.DMA,
        ),
        name="edge_score_sc",
    )


def kernel(x_ligand, x_target, params, edge_index):
    p = params
    n_l, d = x_ligand.shape
    n_t = x_target.shape[0]
    h = p['W_l1_lt'].shape[1]
    e = edge_index.shape[1]
    src = edge_index[0].astype(jnp.int32)
    dst = edge_index[1].astype(jnp.int32)

    chunks = e // _CH
    tpb = ((chunks + _NW * 8 - 1) // (_NW * 8)) * 8
    cpad = _NW * tpb

    seg_t, npad = _make_seg(n_t, d, cpad)
    seg_l, _ = _make_seg(n_l, d, cpad)
    seg_t2, _ = _make_seg(n_t, h, cpad)
    seg_l2, _ = _make_seg(n_l, h, cpad)
    edge = _make_edge(h, cpad, npad)

    def pad3(v, base_row):
        a2 = v.reshape(chunks, _CH)
        padr = jnp.broadcast_to(base_row + jnp.arange(_CH, dtype=jnp.int32),
                                (cpad - chunks, _CH))
        return jnp.concatenate([a2, padr]).reshape(_NW, tpb, _CH)

    src_g = pad3(src, 0)
    dst_g = pad3(dst, 0)
    src_s = pad3(src, n_l)
    dst_s = pad3(dst, n_t)

    def r1(v):
        return v.reshape(1, -1)

    sums_t, cnt_d = seg_t(x_ligand, src_g, dst_s)
    sums_l, cnt_s = seg_l(x_target, dst_g, src_s)
    cnt_d3 = cnt_d[..., None]
    cnt_s3 = cnt_s[..., None]

    t1 = _post(sums_t, cnt_d3, x_target, p['W_l1_lt'], r1(p['b_l1_lt']),
               p['W_r1_lt'], r1(p['gamma1']), r1(p['beta1']))
    l1 = _post(sums_l, cnt_s3, x_ligand, p['W_l1_tl'], r1(p['b_l1_tl']),
               p['W_r1_tl'], r1(p['gamma1']), r1(p['beta1']))

    sums_t2, _unused1 = seg_t2(l1, src_g, dst_s)
    sums_l2, _unused2 = seg_l2(t1, dst_g, src_s)

    bproj = _post_proj(sums_t2, cnt_d3, t1, p['W_l2_lt'], r1(p['b_l2_lt']),
                       p['W_r2_lt'], r1(p['gamma2']), r1(p['beta2']),
                       p['W_e1'][h:], jnp.zeros((1, h), jnp.float32))
    aproj = _post_proj(sums_l2, cnt_s3, l1, p['W_l2_tl'], r1(p['b_l2_tl']),
                       p['W_r2_tl'], r1(p['gamma2']), r1(p['beta2']),
                       p['W_e1'][:h], r1(p['b_e1']), nrows_out=npad)

    wv = p['W_e2'].reshape(-1)
    bv16 = jnp.full((16,), p['b_e2'][0] / 16.0, jnp.float32)
    out = edge(aproj, bproj, src_g, dst_g, wv, bv16)
    return out[:e]


# speedup vs baseline: 1.1981x; 1.1981x over previous
import functools

import jax
import jax.numpy as jnp
from jax import lax
from jax.experimental import pallas as pl
from jax.experimental.pallas import tpu as pltpu
from jax.experimental.pallas import tpu_sc as plsc

_NC = 2
_NS = 16
_NW = _NC * _NS
_CH = 128


def _mesh():
    return plsc.VectorSubcoreMesh(core_axis_name="c", subcore_axis_name="s",
                                  num_cores=_NC, num_subcores=_NS)


def _make_seg(n_nodes, n_feat, n_chunks_pad):
    npad = ((n_nodes + _CH * _NS - 1) // (_CH * _NS)) * (_CH * _NS)
    rows_per_sub = npad // _NS
    blocks_per_sub = rows_per_sub // _CH
    tpb = n_chunks_pad // _NW
    nblk = tpb // 8
    assert nblk % 2 == 0

    def body(x_hbm, gidx_hbm, sidx_hbm, sums_hbm, cnt_hbm,
             gi0, gi1, si0, si1, rows0, rows1, onesbuf, acc_sh, cnt_sh,
             semi0, semi1, semg0, semg1, sems0, sems1):
        c = lax.axis_index("c")
        s = lax.axis_index("s")
        w = s * _NC + c
        gi = (gi0, gi1)
        si = (si0, si1)
        rows = (rows0, rows1)
        semi = (semi0, semi1)
        semg = (semg0, semg1)
        sems = (sems0, sems1)

        def zrow(r, carry):
            for k in range(n_feat // 16):
                rows0[r, pl.ds(k * 16, 16)] = jnp.zeros((16,), jnp.float32)
            return carry
        lax.fori_loop(0, _CH, zrow, 0)
        for k in range(_CH // 16):
            onesbuf[pl.ds(k * 16, 16)] = jnp.ones((16,), jnp.float32)

        for j in range(blocks_per_sub):
            r0 = s * rows_per_sub + j * _CH
            pltpu.sync_copy(rows0, acc_sh.at[pl.ds(r0, _CH)])
            pltpu.sync_copy(rows0.at[0], cnt_sh.at[pl.ds(r0, _CH)])
        plsc.subcore_barrier()

        def fire_idx(slot, blk):
            o = pl.multiple_of(blk * 8, 8)
            pltpu.async_copy(gidx_hbm.at[w, pl.ds(o, 8)], gi[slot],
                             semi[slot])
            pltpu.async_copy(sidx_hbm.at[w, pl.ds(o, 8)], si[slot],
                             semi[slot])

        def drain_idx(slot):
            pltpu.make_async_copy(gidx_hbm.at[0, pl.ds(0, 8)], gi[slot],
                                  semi[slot]).wait()
            pltpu.make_async_copy(sidx_hbm.at[0, pl.ds(0, 8)], si[slot],
                                  semi[slot]).wait()

        def fire_gathers(p, idxrow):
            pltpu.async_copy(x_hbm.at[idxrow], rows[p], semg[p])

        def drain_gathers(p):
            pltpu.make_async_copy(x_hbm.at[gi0.at[0]], rows[p],
                                  semg[p]).wait()

        def fire_scatters(p, idxrow):
            pltpu.async_copy(rows[p], acc_sh.at[idxrow], sems[p], add=True)
            pltpu.async_copy(onesbuf, cnt_sh.at[idxrow], sems[p], add=True)

        def drain_scatters(p):
            pltpu.make_async_copy(rows[p], acc_sh.at[si0.at[0]],
                                  sems[p]).wait()
            pltpu.make_async_copy(onesbuf, cnt_sh.at[si0.at[0]],
                                  sems[p]).wait()

        fire_idx(0, 0)
        drain_idx(0)
        fire_gathers(0, gi0.at[0])

        def step(bb, carry):
            for qq in (0, 1):
                blk = 2 * bb + qq
                giq = gi[qq]
                siq = si[qq]
                for r in range(8):
                    p = r % 2
                    drain_gathers(p)
                    fire_scatters(p, siq.at[r])
                    if r == 0:
                        @pl.when(blk >= 1)
                        def _():
                            drain_scatters(1 - p)

                        @pl.when(blk + 1 < nblk)
                        def _():
                            fire_idx(1 - qq, blk + 1)
                    else:
                        drain_scatters(1 - p)
                    if r < 7:
                        fire_gathers(1 - p, giq.at[r + 1])
                    else:
                        @pl.when(blk + 1 < nblk)
                        def _():
                            drain_idx(1 - qq)
                            fire_gathers(1 - p, gi[1 - qq].at[0])
            return carry
        lax.fori_loop(0, nblk // 2, step, 0)
        drain_scatters(1)
        plsc.subcore_barrier()

        for j in range(blocks_per_sub):
            r0 = s * rows_per_sub + j * _CH
            pltpu.sync_copy(acc_sh.at[pl.ds(r0, _CH)], rows0)
            pltpu.sync_copy(rows0, sums_hbm.at[c, pl.ds(r0, _CH)])
            pltpu.sync_copy(cnt_sh.at[pl.ds(r0, _CH)], onesbuf)
            pltpu.sync_copy(onesbuf, cnt_hbm.at[c, pl.ds(r0, _CH)])

    call = pl.kernel(
        body,
        out_type=(
            jax.ShapeDtypeStruct((_NC, npad, n_feat), jnp.float32),
            jax.ShapeDtypeStruct((_NC, npad), jnp.float32),
        ),
        mesh=_mesh(),
        scratch_types=(
            pltpu.VMEM((8, _CH), jnp.int32),
            pltpu.VMEM((8, _CH), jnp.int32),
            pltpu.VMEM((8, _CH), jnp.int32),
            pltpu.VMEM((8, _CH), jnp.int32),
            pltpu.VMEM((_CH, n_feat), jnp.float32),
            pltpu.VMEM((_CH, n_feat), jnp.float32),
            pltpu.VMEM((_CH,), jnp.float32),
            pltpu.VMEM_SHARED((npad, n_feat), jnp.float32),
            pltpu.VMEM_SHARED((npad,), jnp.float32),
            pltpu.SemaphoreType.DMA,
            pltpu.SemaphoreType.DMA,
            pltpu.SemaphoreType.DMA,
            pltpu.SemaphoreType.DMA,
            pltpu.SemaphoreType.DMA,
            pltpu.SemaphoreType.DMA,
        ),
        name="seg_sum_sc",
    )
    return call, npad


def _post_body(n_nodes, proj, sums_ref, cnt_ref, xdst_ref, wl_ref, bl_ref,
               wr_ref, g_ref, be_ref, *rest):
    if proj:
        wh_ref, bh_ref, out_ref = rest
    else:
        (out_ref,) = rest
    s = sums_ref[0] + sums_ref[1]
    s = s[:n_nodes, :]
    cnt = cnt_ref[0] + cnt_ref[1]
    inv = 1.0 / jnp.maximum(cnt[:n_nodes, :], 1.0)
    mean = s * inv
    y = (jnp.dot(mean, wl_ref[...], preferred_element_type=jnp.float32)
         + bl_ref[...]
         + jnp.dot(xdst_ref[...], wr_ref[...],
                   preferred_element_type=jnp.float32))
    r = jnp.maximum(y, 0.0)
    mu = jnp.mean(r, axis=0, keepdims=True)
    var = jnp.mean((r - mu) * (r - mu), axis=0, keepdims=True)
    feat = (r - mu) * lax.rsqrt(var + 1e-5) * g_ref[...] + be_ref[...]
    if proj:
        out_ref[:n_nodes, :] = (jnp.dot(feat, wh_ref[...],
                                        preferred_element_type=jnp.float32)
                                + bh_ref[...])
    else:
        out_ref[...] = feat


def _post(sums, cnt3, xdst, wl, bl, wr, gam, bet):
    n = xdst.shape[0]
    h = wl.shape[1]
    return pl.pallas_call(
        functools.partial(_post_body, n, False),
        out_shape=jax.ShapeDtypeStruct((n, h), jnp.float32),
    )(sums, cnt3, xdst, wl, bl, wr, gam, bet)


def _post_proj(sums, cnt3, xdst, wl, bl, wr, gam, bet, wh, bh,
               nrows_out=None):
    n = xdst.shape[0]
    h = wh.shape[1]
    return pl.pallas_call(
        functools.partial(_post_body, n, True),
        out_shape=jax.ShapeDtypeStruct((nrows_out or n, h), jnp.float32),
    )(sums, cnt3, xdst, wl, bl, wr, gam, bet, wh, bh)


def _make_edge(n_feat, n_chunks_pad, npad):
    del npad
    tpb = n_chunks_pad // _NW
    assert tpb % 2 == 0

    def body(a_hbm, b_hbm, src_hbm, dst_hbm, w_hbm, bv_hbm, out_hbm,
             sidx_v, didx_v, abuf0, abuf1, bbuf0, bbuf1, wbuf, b16buf,
             outbuf0, outbuf1, semi, sema0, sema1, semb0, semb1,
             semo0, semo1):
        c = lax.axis_index("c")
        s = lax.axis_index("s")
        w = s * _NC + c
        abuf = (abuf0, abuf1)
        bbuf = (bbuf0, bbuf1)
        outbuf = (outbuf0, outbuf1)
        sema = (sema0, sema1)
        semb = (semb0, semb1)
        semo = (semo0, semo1)

        first = w * tpb
        pltpu.async_copy(src_hbm.at[w], sidx_v, semi)
        pltpu.async_copy(dst_hbm.at[w], didx_v, semi)
        pltpu.sync_copy(w_hbm, wbuf)
        pltpu.sync_copy(bv_hbm, b16buf)
        bv = b16buf[...]
        wv = [wbuf[pl.ds(k * 16, 16)] for k in range(n_feat // 16)]
        lanes = lax.iota(jnp.int32, 16)
        pltpu.make_async_copy(src_hbm.at[0], sidx_v, semi).wait()
        pltpu.make_async_copy(dst_hbm.at[0], didx_v, semi).wait()

        def fire(p, i):
            pltpu.async_copy(a_hbm.at[sidx_v.at[i]], abuf[p], sema[p])
            pltpu.async_copy(b_hbm.at[didx_v.at[i]], bbuf[p], semb[p])

        def drain(p):
            pltpu.make_async_copy(a_hbm.at[sidx_v.at[0]], abuf[p],
                                  sema[p]).wait()
            pltpu.make_async_copy(b_hbm.at[didx_v.at[0]], bbuf[p],
                                  semb[p]).wait()

        def compute(ab, bb, ob):
            def group(gi, carry2):
                r = jnp.zeros((16,), jnp.float32)
                for j in range(16):
                    e = gi * 16 + j
                    acc = bv
                    for k in range(n_feat // 16):
                        av = ab[e, pl.ds(k * 16, 16)]
                        bbv = bb[e, pl.ds(k * 16, 16)]
                        acc = acc + jnp.maximum(av + bbv, 0.0) * wv[k]
                    for sh in (8, 4, 2, 1):
                        acc = acc + acc.at[lanes ^ sh].get(
                            mode="promise_in_bounds", unique_indices=True)
                    r = jnp.where(lanes == j, acc, r)
                ob[pl.ds(gi * 16, 16)] = r
                return carry2
            lax.fori_loop(0, _CH // 16, group, 0)

        fire(0, 0)

        def step(g, carry):
            for p in (0, 1):
                i = 2 * g + p
                q = 1 - p
                drain(p)

                @pl.when(i + 1 < tpb)
                def _():
                    fire(q, i + 1)

                @pl.when(i >= 2)
                def _():
                    pltpu.make_async_copy(
                        outbuf[p], out_hbm.at[pl.ds(0, _CH)], semo[p]).wait()

                compute(abuf[p], bbuf[p], outbuf[p])
                pltpu.async_copy(outbuf[p],
                                 out_hbm.at[pl.ds((first + i) * _CH, _CH)],
                                 semo[p])
            return carry
        lax.fori_loop(0, tpb // 2, step, 0)
        for p in (0, 1):
            pltpu.make_async_copy(outbuf[p], out_hbm.at[pl.ds(0, _CH)],
                                  semo[p]).wait()

    return pl.kernel(
        body,
        out_type=jax.ShapeDtypeStruct((n_chunks_pad * _CH,), jnp.float32),
        mesh=_mesh(),
        scratch_types=(
            pltpu.VMEM((tpb, _CH), jnp.int32),
            pltpu.VMEM((tpb, _CH), jnp.int32),
            pltpu.VMEM((_CH, n_feat), jnp.float32),
            pltpu.VMEM((_CH, n_feat), jnp.float32),
            pltpu.VMEM((_CH, n_feat), jnp.float32),
            pltpu.VMEM((_CH, n_feat), jnp.float32),
            pltpu.VMEM((n_feat,), jnp.float32),
            pltpu.VMEM((16,), jnp.float32),
            pltpu.VMEM((_CH,), jnp.float32),
            pltpu.VMEM((_CH,), jnp.float32),
            pltpu.SemaphoreType.DMA,
            pltpu.SemaphoreType.DMA,
            pltpu.SemaphoreType.DMA,
            pltpu.SemaphoreType.DMA,
            pltpu.SemaphoreType.DMA,
            pltpu.SemaphoreType.DMA,
            pltpu.SemaphoreType.DMA,
        ),
        name="edge_score_sc",
    )


def kernel(x_ligand, x_target, params, edge_index):
    p = params
    n_l, d = x_ligand.shape
    n_t = x_target.shape[0]
    h = p['W_l1_lt'].shape[1]
    e = edge_index.shape[1]
    src = edge_index[0].astype(jnp.int32)
    dst = edge_index[1].astype(jnp.int32)

    chunks = e // _CH
    tpb = ((chunks + _NW * 8 - 1) // (_NW * 8)) * 8
    cpad = _NW * tpb

    seg_t, npad = _make_seg(n_t, d, cpad)
    seg_l, _ = _make_seg(n_l, d, cpad)
    seg_t2, _ = _make_seg(n_t, h, cpad)
    seg_l2, _ = _make_seg(n_l, h, cpad)
    edge = _make_edge(h, cpad, npad)

    def pad3(v, base_row):
        a2 = v.reshape(chunks, _CH)
        padr = jnp.broadcast_to(base_row + jnp.arange(_CH, dtype=jnp.int32),
                                (cpad - chunks, _CH))
        return jnp.concatenate([a2, padr]).reshape(_NW, tpb, _CH)

    src_g = pad3(src, 0)
    dst_g = pad3(dst, 0)
    src_s = pad3(src, n_l)
    dst_s = pad3(dst, n_t)

    def r1(v):
        return v.reshape(1, -1)

    sums_t, cnt_d = seg_t(x_ligand, src_g, dst_s)
    sums_l, cnt_s = seg_l(x_target, dst_g, src_s)
    cnt_d3 = cnt_d[..., None]
    cnt_s3 = cnt_s[..., None]

    t1 = _post(sums_t, cnt_d3, x_target, p['W_l1_lt'], r1(p['b_l1_lt']),
               p['W_r1_lt'], r1(p['gamma1']), r1(p['beta1']))
    l1 = _post(sums_l, cnt_s3, x_ligand, p['W_l1_tl'], r1(p['b_l1_tl']),
               p['W_r1_tl'], r1(p['gamma1']), r1(p['beta1']))

    sums_t2, _unused1 = seg_t2(l1, src_g, dst_s)
    sums_l2, _unused2 = seg_l2(t1, dst_g, src_s)

    bproj = _post_proj(sums_t2, cnt_d3, t1, p['W_l2_lt'], r1(p['b_l2_lt']),
                       p['W_r2_lt'], r1(p['gamma2']), r1(p['beta2']),
                       p['W_e1'][h:], jnp.zeros((1, h), jnp.float32))
    aproj = _post_proj(sums_l2, cnt_s3, l1, p['W_l2_tl'], r1(p['b_l2_tl']),
                       p['W_r2_tl'], r1(p['gamma2']), r1(p['beta2']),
                       p['W_e1'][:h], r1(p['b_e1']), nrows_out=npad)

    wv = p['W_e2'].reshape(-1)
    bv16 = jnp.full((16,), p['b_e2'][0] / 16.0, jnp.float32)
    out = edge(aproj, bproj, src_g, dst_g, wv, bv16)
    return out[:e]


# speedup vs baseline: 1.2047x; 1.0056x over previous
import functools

import jax
import jax.numpy as jnp
from jax import lax
from jax.experimental import pallas as pl
from jax.experimental.pallas import tpu as pltpu
from jax.experimental.pallas import tpu_sc as plsc

_NC = 2
_NS = 16
_NW = _NC * _NS
_CH = 128


def _mesh():
    return plsc.VectorSubcoreMesh(core_axis_name="c", subcore_axis_name="s",
                                  num_cores=_NC, num_subcores=_NS)


def _make_seg(n_nodes, n_feat, n_chunks_pad, with_counts=True):
    npad = ((n_nodes + _CH * _NS - 1) // (_CH * _NS)) * (_CH * _NS)
    rows_per_sub = npad // _NS
    blocks_per_sub = rows_per_sub // _CH
    tpb = n_chunks_pad // _NW
    nblk = tpb // 8
    assert nblk % 2 == 0

    def body(x_hbm, gidx_hbm, sidx_hbm, sums_hbm, *rest):
        if with_counts:
            cnt_hbm = rest[0]
            rest = rest[1:]
        else:
            cnt_hbm = None
        (gi0, gi1, si0, si1, rows0, rows1, onesbuf, acc_sh, cnt_sh,
         semi0, semi1, semg0, semg1, sems0, sems1) = rest
        c = lax.axis_index("c")
        s = lax.axis_index("s")
        w = s * _NC + c
        gi = (gi0, gi1)
        si = (si0, si1)
        rows = (rows0, rows1)
        semi = (semi0, semi1)
        semg = (semg0, semg1)
        sems = (sems0, sems1)

        def zrow(r, carry):
            for k in range(n_feat // 16):
                rows0[r, pl.ds(k * 16, 16)] = jnp.zeros((16,), jnp.float32)
            return carry
        lax.fori_loop(0, _CH, zrow, 0)
        for k in range(_CH // 16):
            onesbuf[pl.ds(k * 16, 16)] = jnp.ones((16,), jnp.float32)

        for j in range(blocks_per_sub):
            r0 = s * rows_per_sub + j * _CH
            pltpu.async_copy(rows0, acc_sh.at[pl.ds(r0, _CH)], semg0)
            if with_counts:
                pltpu.async_copy(rows0.at[0], cnt_sh.at[pl.ds(r0, _CH)],
                                 semg0)
        for j in range(blocks_per_sub):
            pltpu.make_async_copy(rows0, acc_sh.at[pl.ds(0, _CH)],
                                  semg0).wait()
            if with_counts:
                pltpu.make_async_copy(rows0.at[0], cnt_sh.at[pl.ds(0, _CH)],
                                      semg0).wait()
        plsc.subcore_barrier()

        def fire_idx(slot, blk):
            o = pl.multiple_of(blk * 8, 8)
            pltpu.async_copy(gidx_hbm.at[w, pl.ds(o, 8)], gi[slot],
                             semi[slot])
            pltpu.async_copy(sidx_hbm.at[w, pl.ds(o, 8)], si[slot],
                             semi[slot])

        def drain_idx(slot):
            pltpu.make_async_copy(gidx_hbm.at[0, pl.ds(0, 8)], gi[slot],
                                  semi[slot]).wait()
            pltpu.make_async_copy(sidx_hbm.at[0, pl.ds(0, 8)], si[slot],
                                  semi[slot]).wait()

        def fire_gathers(p, idxrow):
            pltpu.async_copy(x_hbm.at[idxrow], rows[p], semg[p])

        def drain_gathers(p):
            pltpu.make_async_copy(x_hbm.at[gi0.at[0]], rows[p],
                                  semg[p]).wait()

        def fire_scatters(p, idxrow):
            pltpu.async_copy(rows[p], acc_sh.at[idxrow], sems[p], add=True)
            if with_counts:
                pltpu.async_copy(onesbuf, cnt_sh.at[idxrow], sems[p],
                                 add=True)

        def drain_scatters(p):
            pltpu.make_async_copy(rows[p], acc_sh.at[si0.at[0]],
                                  sems[p]).wait()
            if with_counts:
                pltpu.make_async_copy(onesbuf, cnt_sh.at[si0.at[0]],
                                      sems[p]).wait()

        fire_idx(0, 0)
        drain_idx(0)
        fire_gathers(0, gi0.at[0])

        def step(bb, carry):
            for qq in (0, 1):
                blk = 2 * bb + qq
                giq = gi[qq]
                siq = si[qq]
                for r in range(8):
                    p = r % 2
                    drain_gathers(p)
                    fire_scatters(p, siq.at[r])
                    if r == 0:
                        @pl.when(blk >= 1)
                        def _():
                            drain_scatters(1 - p)

                        @pl.when(blk + 1 < nblk)
                        def _():
                            fire_idx(1 - qq, blk + 1)
                    else:
                        drain_scatters(1 - p)
                    if r < 7:
                        fire_gathers(1 - p, giq.at[r + 1])
                    else:
                        @pl.when(blk + 1 < nblk)
                        def _():
                            drain_idx(1 - qq)
                            fire_gathers(1 - p, gi[1 - qq].at[0])
            return carry
        lax.fori_loop(0, nblk // 2, step, 0)
        drain_scatters(1)
        plsc.subcore_barrier()

        for j in range(blocks_per_sub):
            p = j % 2
            r0 = s * rows_per_sub + j * _CH
            if j >= 2:
                pltpu.make_async_copy(rows[p],
                                      sums_hbm.at[0, pl.ds(0, _CH)],
                                      semg[p]).wait()
            pltpu.sync_copy(acc_sh.at[pl.ds(r0, _CH)], rows[p])
            pltpu.async_copy(rows[p], sums_hbm.at[c, pl.ds(r0, _CH)],
                             semg[p])
        for j in range(max(blocks_per_sub - 2, 0), blocks_per_sub):
            pltpu.make_async_copy(rows[j % 2],
                                  sums_hbm.at[0, pl.ds(0, _CH)],
                                  semg[j % 2]).wait()
        if with_counts:
            for j in range(blocks_per_sub):
                r0 = s * rows_per_sub + j * _CH
                pltpu.sync_copy(cnt_sh.at[pl.ds(r0, _CH)], onesbuf)
                pltpu.sync_copy(onesbuf, cnt_hbm.at[c, pl.ds(r0, _CH)])

    out_type = [jax.ShapeDtypeStruct((_NC, npad, n_feat), jnp.float32)]
    if with_counts:
        out_type.append(jax.ShapeDtypeStruct((_NC, npad), jnp.float32))
    call = pl.kernel(
        body,
        out_type=tuple(out_type),
        mesh=_mesh(),
        scratch_types=(
            pltpu.VMEM((8, _CH), jnp.int32),
            pltpu.VMEM((8, _CH), jnp.int32),
            pltpu.VMEM((8, _CH), jnp.int32),
            pltpu.VMEM((8, _CH), jnp.int32),
            pltpu.VMEM((_CH, n_feat), jnp.float32),
            pltpu.VMEM((_CH, n_feat), jnp.float32),
            pltpu.VMEM((_CH,), jnp.float32),
            pltpu.VMEM_SHARED((npad, n_feat), jnp.float32),
            pltpu.VMEM_SHARED((npad,), jnp.float32),
            pltpu.SemaphoreType.DMA,
            pltpu.SemaphoreType.DMA,
            pltpu.SemaphoreType.DMA,
            pltpu.SemaphoreType.DMA,
            pltpu.SemaphoreType.DMA,
            pltpu.SemaphoreType.DMA,
        ),
        name="seg_sum_sc",
    )
    return call, npad


def _post_body(n_nodes, proj, sums_ref, cnt_ref, xdst_ref, wl_ref, bl_ref,
               wr_ref, g_ref, be_ref, *rest):
    if proj:
        wh_ref, bh_ref, out_ref = rest
    else:
        (out_ref,) = rest
    s = sums_ref[0] + sums_ref[1]
    s = s[:n_nodes, :]
    cnt = cnt_ref[0] + cnt_ref[1]
    inv = 1.0 / jnp.maximum(cnt[:n_nodes, :], 1.0)
    mean = s * inv
    y = (jnp.dot(mean, wl_ref[...], preferred_element_type=jnp.float32)
         + bl_ref[...]
         + jnp.dot(xdst_ref[...], wr_ref[...],
                   preferred_element_type=jnp.float32))
    r = jnp.maximum(y, 0.0)
    mu = jnp.mean(r, axis=0, keepdims=True)
    var = jnp.mean((r - mu) * (r - mu), axis=0, keepdims=True)
    feat = (r - mu) * lax.rsqrt(var + 1e-5) * g_ref[...] + be_ref[...]
    if proj:
        out_ref[:n_nodes, :] = (jnp.dot(feat, wh_ref[...],
                                        preferred_element_type=jnp.float32)
                                + bh_ref[...])
    else:
        out_ref[...] = feat


def _post(sums, cnt3, xdst, wl, bl, wr, gam, bet):
    n = xdst.shape[0]
    h = wl.shape[1]
    return pl.pallas_call(
        functools.partial(_post_body, n, False),
        out_shape=jax.ShapeDtypeStruct((n, h), jnp.float32),
    )(sums, cnt3, xdst, wl, bl, wr, gam, bet)


def _post_proj(sums, cnt3, xdst, wl, bl, wr, gam, bet, wh, bh,
               nrows_out=None):
    n = xdst.shape[0]
    h = wh.shape[1]
    return pl.pallas_call(
        functools.partial(_post_body, n, True),
        out_shape=jax.ShapeDtypeStruct((nrows_out or n, h), jnp.float32),
    )(sums, cnt3, xdst, wl, bl, wr, gam, bet, wh, bh)


def _make_edge(n_feat, n_chunks_pad, npad):
    del npad
    tpb = n_chunks_pad // _NW
    assert tpb % 2 == 0

    def body(a_hbm, b_hbm, src_hbm, dst_hbm, w_hbm, bv_hbm, out_hbm,
             sidx_v, didx_v, abuf0, abuf1, bbuf0, bbuf1, wbuf, b16buf,
             outbuf0, outbuf1, semi, sema0, sema1, semb0, semb1,
             semo0, semo1):
        c = lax.axis_index("c")
        s = lax.axis_index("s")
        w = s * _NC + c
        abuf = (abuf0, abuf1)
        bbuf = (bbuf0, bbuf1)
        outbuf = (outbuf0, outbuf1)
        sema = (sema0, sema1)
        semb = (semb0, semb1)
        semo = (semo0, semo1)

        first = w * tpb
        pltpu.async_copy(src_hbm.at[w], sidx_v, semi)
        pltpu.async_copy(dst_hbm.at[w], didx_v, semi)
        pltpu.sync_copy(w_hbm, wbuf)
        pltpu.sync_copy(bv_hbm, b16buf)
        bv = b16buf[...]
        wv = [wbuf[pl.ds(k * 16, 16)] for k in range(n_feat // 16)]
        lanes = lax.iota(jnp.int32, 16)
        pltpu.make_async_copy(src_hbm.at[0], sidx_v, semi).wait()
        pltpu.make_async_copy(dst_hbm.at[0], didx_v, semi).wait()

        def fire(p, i):
            pltpu.async_copy(a_hbm.at[sidx_v.at[i]], abuf[p], sema[p])
            pltpu.async_copy(b_hbm.at[didx_v.at[i]], bbuf[p], semb[p])

        def drain(p):
            pltpu.make_async_copy(a_hbm.at[sidx_v.at[0]], abuf[p],
                                  sema[p]).wait()
            pltpu.make_async_copy(b_hbm.at[didx_v.at[0]], bbuf[p],
                                  semb[p]).wait()

        def compute(ab, bb, ob):
            def group(gi, carry2):
                r = jnp.zeros((16,), jnp.float32)
                for j in range(16):
                    e = gi * 16 + j
                    acc = bv
                    for k in range(n_feat // 16):
                        av = ab[e, pl.ds(k * 16, 16)]
                        bbv = bb[e, pl.ds(k * 16, 16)]
                        acc = acc + jnp.maximum(av + bbv, 0.0) * wv[k]
                    for sh in (8, 4, 2, 1):
                        acc = acc + acc.at[lanes ^ sh].get(
                            mode="promise_in_bounds", unique_indices=True)
                    r = jnp.where(lanes == j, acc, r)
                ob[pl.ds(gi * 16, 16)] = r
                return carry2
            lax.fori_loop(0, _CH // 16, group, 0)

        fire(0, 0)

        def step(g, carry):
            for p in (0, 1):
                i = 2 * g + p
                q = 1 - p
                drain(p)

                @pl.when(i + 1 < tpb)
                def _():
                    fire(q, i + 1)

                @pl.when(i >= 2)
                def _():
                    pltpu.make_async_copy(
                        outbuf[p], out_hbm.at[pl.ds(0, _CH)], semo[p]).wait()

                compute(abuf[p], bbuf[p], outbuf[p])
                pltpu.async_copy(outbuf[p],
                                 out_hbm.at[pl.ds((first + i) * _CH, _CH)],
                                 semo[p])
            return carry
        lax.fori_loop(0, tpb // 2, step, 0)
        for p in (0, 1):
            pltpu.make_async_copy(outbuf[p], out_hbm.at[pl.ds(0, _CH)],
                                  semo[p]).wait()

    return pl.kernel(
        body,
        out_type=jax.ShapeDtypeStruct((n_chunks_pad * _CH,), jnp.float32),
        mesh=_mesh(),
        scratch_types=(
            pltpu.VMEM((tpb, _CH), jnp.int32),
            pltpu.VMEM((tpb, _CH), jnp.int32),
            pltpu.VMEM((_CH, n_feat), jnp.float32),
            pltpu.VMEM((_CH, n_feat), jnp.float32),
            pltpu.VMEM((_CH, n_feat), jnp.float32),
            pltpu.VMEM((_CH, n_feat), jnp.float32),
            pltpu.VMEM((n_feat,), jnp.float32),
            pltpu.VMEM((16,), jnp.float32),
            pltpu.VMEM((_CH,), jnp.float32),
            pltpu.VMEM((_CH,), jnp.float32),
            pltpu.SemaphoreType.DMA,
            pltpu.SemaphoreType.DMA,
            pltpu.SemaphoreType.DMA,
            pltpu.SemaphoreType.DMA,
            pltpu.SemaphoreType.DMA,
            pltpu.SemaphoreType.DMA,
            pltpu.SemaphoreType.DMA,
        ),
        name="edge_score_sc",
    )


def kernel(x_ligand, x_target, params, edge_index):
    p = params
    n_l, d = x_ligand.shape
    n_t = x_target.shape[0]
    h = p['W_l1_lt'].shape[1]
    e = edge_index.shape[1]
    src = edge_index[0].astype(jnp.int32)
    dst = edge_index[1].astype(jnp.int32)

    chunks = e // _CH
    tpb = ((chunks + _NW * 8 - 1) // (_NW * 8)) * 8
    cpad = _NW * tpb

    seg_t, npad = _make_seg(n_t, d, cpad)
    seg_l, _ = _make_seg(n_l, d, cpad)
    seg_t2, _ = _make_seg(n_t, h, cpad, with_counts=False)
    seg_l2, _ = _make_seg(n_l, h, cpad, with_counts=False)
    edge = _make_edge(h, cpad, npad)

    def pad3(v, base_row):
        a2 = v.reshape(chunks, _CH)
        padr = jnp.broadcast_to(base_row + jnp.arange(_CH, dtype=jnp.int32),
                                (cpad - chunks, _CH))
        return jnp.concatenate([a2, padr]).reshape(_NW, tpb, _CH)

    src_g = pad3(src, 0)
    dst_g = pad3(dst, 0)
    src_s = pad3(src, n_l)
    dst_s = pad3(dst, n_t)

    def r1(v):
        return v.reshape(1, -1)

    sums_t, cnt_d = seg_t(x_ligand, src_g, dst_s)
    sums_l, cnt_s = seg_l(x_target, dst_g, src_s)
    cnt_d3 = cnt_d[..., None]
    cnt_s3 = cnt_s[..., None]

    t1 = _post(sums_t, cnt_d3, x_target, p['W_l1_lt'], r1(p['b_l1_lt']),
               p['W_r1_lt'], r1(p['gamma1']), r1(p['beta1']))
    l1 = _post(sums_l, cnt_s3, x_ligand, p['W_l1_tl'], r1(p['b_l1_tl']),
               p['W_r1_tl'], r1(p['gamma1']), r1(p['beta1']))

    (sums_t2,) = seg_t2(l1, src_g, dst_s)
    (sums_l2,) = seg_l2(t1, dst_g, src_s)

    bproj = _post_proj(sums_t2, cnt_d3, t1, p['W_l2_lt'], r1(p['b_l2_lt']),
                       p['W_r2_lt'], r1(p['gamma2']), r1(p['beta2']),
                       p['W_e1'][h:], jnp.zeros((1, h), jnp.float32))
    aproj = _post_proj(sums_l2, cnt_s3, l1, p['W_l2_tl'], r1(p['b_l2_tl']),
                       p['W_r2_tl'], r1(p['gamma2']), r1(p['beta2']),
                       p['W_e1'][:h], r1(p['b_e1']), nrows_out=npad)

    wv = p['W_e2'].reshape(-1)
    bv16 = jnp.full((16,), p['b_e2'][0] / 16.0, jnp.float32)
    out = edge(aproj, bproj, src_g, dst_g, wv, bv16)
    return out[:e]
